# Initial kernel scaffold; baseline (speedup 1.0000x reference)
import jax
import jax.numpy as jnp
from jax.experimental import pallas as pl


def kernel(edge_index, atom_types, edge_energy, scales, shifts):
    raise NotImplementedError("write your pallas kernel here")


# speedup vs baseline: 892.6370x; 892.6370x over previous
import functools

import jax
import jax.numpy as jnp
from jax import lax
from jax.experimental import pallas as pl
from jax.experimental.pallas import tpu as pltpu
from jax.experimental.pallas import tpu_sc as plsc

N_NODES_K = 100000
N_EDGES_K = 6400000
NUM_TYPES_K = 64
TBL = NUM_TYPES_K * NUM_TYPES_K

LANES = 16
N_WORKERS = 32
E_PER_W = N_EDGES_K // N_WORKERS
CHUNK = 2000
N_CHUNKS = E_PER_W // CHUNK


def _make_body():
    mesh = plsc.VectorSubcoreMesh(core_axis_name="c", subcore_axis_name="s")

    @functools.partial(
        pl.kernel,
        out_type=jax.ShapeDtypeStruct((N_EDGES_K,), jnp.float32),
        mesh=mesh,
        compiler_params=pltpu.CompilerParams(needs_layout_passes=False),
        scratch_types=[
            pltpu.VMEM((N_NODES_K,), jnp.int32),
            pltpu.VMEM((TBL,), jnp.int32),
            pltpu.VMEM((CHUNK,), jnp.int32),
            pltpu.VMEM((CHUNK,), jnp.int32),
            pltpu.VMEM((CHUNK,), jnp.float32),
            pltpu.VMEM((CHUNK,), jnp.int32),
            pltpu.VMEM((CHUNK,), jnp.int32),
            pltpu.VMEM((CHUNK,), jnp.float32),
            pltpu.VMEM((CHUNK,), jnp.float32),
            pltpu.SemaphoreType.DMA,
            pltpu.SemaphoreType.DMA,
        ],
    )
    def body(idx_c_hbm, idx_n_hbm, energy_hbm, types_hbm, packed_hbm,
             out_hbm, types_v, packed_v, icv0, inv0, ev0, icv1, inv1, ev1,
             ov, sem0, sem1):
        cid = lax.axis_index("c")
        sid = lax.axis_index("s")
        wid = sid * 2 + cid
        base_w = wid * E_PER_W

        pltpu.sync_copy(types_hbm, types_v)
        pltpu.sync_copy(packed_hbm, packed_v)

        banks = ((icv0, inv0, ev0, sem0), (icv1, inv1, ev1, sem1))

        def start_loads(j, b):
            icv, inv, ev, sem = banks[b]
            base = base_w + j * CHUNK
            pltpu.async_copy(idx_c_hbm.at[pl.ds(base, CHUNK)], icv, sem)
            pltpu.async_copy(idx_n_hbm.at[pl.ds(base, CHUNK)], inv, sem)
            pltpu.async_copy(energy_hbm.at[pl.ds(base, CHUNK)], ev, sem)

        def wait_loads(b):
            icv, inv, ev, sem = banks[b]
            pltpu.make_async_copy(idx_c_hbm.at[pl.ds(0, CHUNK)], icv, sem).wait()
            pltpu.make_async_copy(idx_n_hbm.at[pl.ds(0, CHUNK)], inv, sem).wait()
            pltpu.make_async_copy(energy_hbm.at[pl.ds(0, CHUNK)], ev, sem).wait()

        hi_mask = jnp.int32(-65536)

        def compute(j, b):
            icv, inv, ev, _ = banks[b]

            def vec(i, c2):
                s = i * LANES
                ic = icv[pl.ds(s, LANES)]
                in_ = inv[pl.ds(s, LANES)]
                tc = plsc.load_gather(types_v, [ic])
                tn = plsc.load_gather(types_v, [in_])
                pair = tc * NUM_TYPES_K + tn
                pk = plsc.load_gather(packed_v, [pair])
                scl = plsc.bitcast(pk & hi_mask, jnp.float32)
                sh = plsc.bitcast(pk << 16, jnp.float32)
                e = ev[pl.ds(s, LANES)]
                ov[pl.ds(s, LANES)] = scl * e + sh
                return c2

            lax.fori_loop(0, CHUNK // LANES, vec, 0)
            pltpu.sync_copy(ov, out_hbm.at[pl.ds(base_w + j * CHUNK, CHUNK)])

        start_loads(0, 0)

        def outer(i, carry):
            j0 = i * 2
            start_loads(j0 + 1, 1)
            wait_loads(0)
            compute(j0, 0)

            @pl.when(j0 + 2 < N_CHUNKS)
            def _():
                start_loads(j0 + 2, 0)

            wait_loads(1)
            compute(j0 + 1, 1)
            return carry

        lax.fori_loop(0, N_CHUNKS // 2, outer, 0)

    return body


_body = _make_body()


def kernel(edge_index, atom_types, edge_energy, scales, shifts):
    s16 = jax.lax.bitcast_convert_type(
        scales.astype(jnp.bfloat16), jnp.uint16).astype(jnp.uint32)
    h16 = jax.lax.bitcast_convert_type(
        shifts.astype(jnp.bfloat16), jnp.uint16).astype(jnp.uint32)
    packed = ((s16 << 16) | h16).astype(jnp.int32).reshape(-1)
    out = _body(
        edge_index[0],
        edge_index[1],
        edge_energy.reshape(-1),
        atom_types,
        packed,
    )
    return out.reshape(-1, 1)
